# Initial kernel scaffold; baseline (speedup 1.0000x reference)
import jax
import jax.numpy as jnp
from jax.experimental import pallas as pl


def kernel(claim_embeddings, x, edge_index, node_batch, W1, att_src1, att_dst1, b1, W2, att_src2, att_dst2, b2, bn_gamma, bn_beta, Wc, bc):
    raise NotImplementedError("write your pallas kernel here")


# speedup vs baseline: 19.6279x; 19.6279x over previous
import functools
import jax
import jax.numpy as jnp
from jax import lax
from jax.experimental import pallas as pl
from jax.experimental.pallas import tpu as pltpu
from jax.experimental.pallas import tpu_sc as plsc

N = 10000
NP = 10240
BG = 32
DIN = 768
DH = 128
E_RAW = 320000
E = E_RAW + N
NC, NS, L = 2, 16, 16
NW = NC * NS
CH = 128
NCHUNK = 81
ET = NCHUNK * CH
EP = NW * ET
STRIP = NP // NS
RB = 16
RBS = NP // RB
HIGH = lax.Precision.HIGHEST


def _tc1_body(x_ref, nb_ref, claim_ref, w1_ref, avs_ref, avd_ref,
              h_ref, as_ref, ad_ref):
    x = x_ref[...]
    nb = nb_ref[0, 0, :].reshape(RBS, 1)
    gid = lax.broadcasted_iota(jnp.int32, (RBS, BG), 1)
    onehot = (nb == gid).astype(jnp.float32)
    claim = claim_ref[...]
    ce = lax.dot_general(onehot, claim, (((1,), (0,)), ((), ())),
                         preferred_element_type=jnp.float32, precision=HIGH)
    dot = jnp.sum(ce * x, axis=1)
    na = jnp.sqrt(jnp.sum(ce * ce, axis=1))
    nx = jnp.sqrt(jnp.sum(x * x, axis=1))
    rel = dot / jnp.maximum(na * nx, 1e-8)
    h0 = x * rel[:, None]
    h1 = lax.dot_general(h0, w1_ref[...], (((1,), (0,)), ((), ())),
                         preferred_element_type=jnp.float32, precision=HIGH)
    h_ref[...] = h1
    as_ref[0, 0, :] = jnp.sum(h1 * avs_ref[...], axis=1)
    ad_ref[0, 0, :] = jnp.sum(h1 * avd_ref[...], axis=1)


def _tc1(x_p, nb3, claim, W1, att_src, att_dst):
    return pl.pallas_call(
        _tc1_body,
        grid=(RB,),
        in_specs=[
            pl.BlockSpec((RBS, DIN), lambda i: (i, 0)),
            pl.BlockSpec((1, 1, RBS), lambda i: (i, 0, 0)),
            pl.BlockSpec((BG, DIN), lambda i: (0, 0)),
            pl.BlockSpec((DIN, DH), lambda i: (0, 0)),
            pl.BlockSpec((1, DH), lambda i: (0, 0)),
            pl.BlockSpec((1, DH), lambda i: (0, 0)),
        ],
        out_specs=[
            pl.BlockSpec((RBS, DH), lambda i: (i, 0)),
            pl.BlockSpec((1, 1, RBS), lambda i: (i, 0, 0)),
            pl.BlockSpec((1, 1, RBS), lambda i: (i, 0, 0)),
        ],
        out_shape=[
            jax.ShapeDtypeStruct((NP, DH), jnp.float32),
            jax.ShapeDtypeStruct((RB, 1, RBS), jnp.float32),
            jax.ShapeDtypeStruct((RB, 1, RBS), jnp.float32),
        ],
    )(x_p, nb3, claim, W1, att_src, att_dst)


def _sca_body(src_h, dst_h, as_h, ad_h, ee_h, dpart_h,
              asrc_v, adst_v, srcv, dstv, eev, zbuf, denom_sp):
    cid = lax.axis_index("c")
    sid = lax.axis_index("s")
    wid = sid * NC + cid
    pltpu.sync_copy(as_h, asrc_v)
    pltpu.sync_copy(ad_h, adst_v)
    pltpu.sync_copy(src_h.at[wid], srcv)
    pltpu.sync_copy(dst_h.at[wid], dstv)
    z = jnp.zeros((L,), jnp.float32)

    def zb(i, carry):
        zbuf[pl.ds(i * L, L)] = z
        return carry
    lax.fori_loop(0, STRIP // L, zb, 0)
    pltpu.sync_copy(zbuf, denom_sp.at[pl.ds(sid * STRIP, STRIP)])
    plsc.subcore_barrier()

    def body(c, carry):
        for j in range(CH // L):
            s16 = srcv[c, pl.ds(j * L, L)]
            d16 = dstv[c, pl.ds(j * L, L)]
            t = plsc.load_gather(asrc_v, [s16]) + plsc.load_gather(adst_v, [d16])
            e = jnp.maximum(t, 0.2 * t)
            eev[c, pl.ds(j * L, L)] = jnp.exp(e)
        pltpu.sync_copy(eev.at[c], denom_sp.at[dstv.at[c]], add=True)
        return carry
    lax.fori_loop(0, NCHUNK, body, 0)
    pltpu.sync_copy(eev, ee_h.at[wid])
    plsc.subcore_barrier()
    pltpu.sync_copy(denom_sp.at[pl.ds(sid * STRIP, STRIP)],
                    dpart_h.at[cid, pl.ds(sid * STRIP, STRIP)])


def _sca(src2, dst2, a_src, a_dst):
    mesh = plsc.VectorSubcoreMesh(core_axis_name="c", subcore_axis_name="s", num_cores=NC, num_subcores=NS)
    f = pl.kernel(
        _sca_body,
        out_type=[
            jax.ShapeDtypeStruct((NW, NCHUNK, CH), jnp.float32),
            jax.ShapeDtypeStruct((NC, NP), jnp.float32),
        ],
        mesh=mesh,
        compiler_params=pltpu.CompilerParams(needs_layout_passes=False),
        scratch_types=[
            pltpu.VMEM((NP,), jnp.float32),
            pltpu.VMEM((NP,), jnp.float32),
            pltpu.VMEM((NCHUNK, CH), jnp.int32),
            pltpu.VMEM((NCHUNK, CH), jnp.int32),
            pltpu.VMEM((NCHUNK, CH), jnp.float32),
            pltpu.VMEM((STRIP,), jnp.float32),
            pltpu.VMEM_SHARED((NP,), jnp.float32),
        ],
    )
    return f(src2, dst2, a_src, a_dst)


def _scb_body(src_h, dst_h, ee_h, dp_h, h_h, part_h,
              dstv, inv_v, d1s, rows, alphav, src_pc, ee_pc,
              sem, out_sp):
    cid = lax.axis_index("c")
    sid = lax.axis_index("s")
    wid = sid * NC + cid
    pltpu.sync_copy(dst_h.at[wid], dstv)
    pltpu.sync_copy(dp_h.at[0], inv_v)

    def invt(t, carry):
        pltpu.sync_copy(dp_h.at[1].at[pl.ds(t * STRIP, STRIP)], d1s)

        def invb(k, c2):
            s = pl.ds(t * STRIP + k * L, L)
            inv_v[s] = 1.0 / (inv_v[s] + d1s[pl.ds(k * L, L)] + 1e-16)
            return c2
        lax.fori_loop(0, STRIP // L, invb, 0)
        return carry
    lax.fori_loop(0, NS, invt, 0)

    z = jnp.zeros((L,), jnp.float32)

    def zrow(i, carry):
        for j in range(DH // L):
            rows[i, pl.ds(j * L, L)] = z
        return carry
    lax.fori_loop(0, CH, zrow, 0)

    def zstrip(k, carry):
        pltpu.sync_copy(rows, out_sp.at[pl.ds(sid * STRIP + k * CH, CH)])
        return carry
    lax.fori_loop(0, STRIP // CH, zstrip, 0)
    plsc.subcore_barrier()

    def body(c, carry):
        pltpu.sync_copy(src_h.at[wid].at[c], src_pc)
        pltpu.sync_copy(ee_h.at[wid].at[c], ee_pc)
        pltpu.async_copy(h_h.at[src_pc], rows, sem).wait()
        for j in range(CH // L):
            d16 = dstv[c, pl.ds(j * L, L)]
            a16 = ee_pc[pl.ds(j * L, L)] * plsc.load_gather(inv_v, [d16])
            alphav[pl.ds(j * L, L)] = a16

        def scale(jj, c2):
            a16 = alphav[pl.ds(jj * L, L)]
            for k in range(L):
                s = a16[k]
                i = jj * L + k
                for j in range(DH // L):
                    sl = pl.ds(j * L, L)
                    rows[i, sl] = rows[i, sl] * s
            return c2
        lax.fori_loop(0, CH // L, scale, 0)
        pltpu.sync_copy(rows, out_sp.at[dstv.at[c]], add=True)
        return carry
    lax.fori_loop(0, NCHUNK, body, 0)
    plsc.subcore_barrier()
    pltpu.sync_copy(out_sp.at[pl.ds(sid * STRIP, STRIP)],
                    part_h.at[cid, pl.ds(sid * STRIP, STRIP)])


def _scb(src2, dst2, ee, dparts, h):
    mesh = plsc.VectorSubcoreMesh(core_axis_name="c", subcore_axis_name="s", num_cores=NC, num_subcores=NS)
    f = pl.kernel(
        _scb_body,
        out_type=jax.ShapeDtypeStruct((NC, NP, DH), jnp.float32),
        mesh=mesh,
        compiler_params=pltpu.CompilerParams(needs_layout_passes=False),
        scratch_types=[
            pltpu.VMEM((NCHUNK, CH), jnp.int32),
            pltpu.VMEM((NP,), jnp.float32),
            pltpu.VMEM((STRIP,), jnp.float32),
            pltpu.VMEM((CH, DH), jnp.float32),
            pltpu.VMEM((CH,), jnp.float32),
            pltpu.VMEM((CH,), jnp.int32),
            pltpu.VMEM((CH,), jnp.float32),
            pltpu.SemaphoreType.DMA,
            pltpu.VMEM_SHARED((NP, DH), jnp.float32),
        ],
    )
    return f(src2, dst2, ee, dparts, h)


def _tc2a_body(p_ref, b_ref, o_ref, s_ref, q_ref):
    i = pl.program_id(0)
    o = p_ref[0] + p_ref[1] + b_ref[...]
    o_ref[...] = o
    rid = lax.broadcasted_iota(jnp.int32, (RBS, 1), 0) + i * RBS
    m = (rid < N).astype(jnp.float32)
    om = o * m
    s_ref[0, :, :] = jnp.sum(om, axis=0, keepdims=True)
    q_ref[0, :, :] = jnp.sum(om * o, axis=0, keepdims=True)


def _tc2a(parts, b1):
    return pl.pallas_call(
        _tc2a_body,
        grid=(RB,),
        in_specs=[
            pl.BlockSpec((2, RBS, DH), lambda i: (0, i, 0)),
            pl.BlockSpec((1, DH), lambda i: (0, 0)),
        ],
        out_specs=[
            pl.BlockSpec((RBS, DH), lambda i: (i, 0)),
            pl.BlockSpec((1, 1, DH), lambda i: (i, 0, 0)),
            pl.BlockSpec((1, 1, DH), lambda i: (i, 0, 0)),
        ],
        out_shape=[
            jax.ShapeDtypeStruct((NP, DH), jnp.float32),
            jax.ShapeDtypeStruct((RB, 1, DH), jnp.float32),
            jax.ShapeDtypeStruct((RB, 1, DH), jnp.float32),
        ],
    )(parts, b1)


def _tc2b_body(o_ref, mu_ref, is_ref, g_ref, be_ref, w2_ref, avs_ref, avd_ref,
               h_ref, as_ref, ad_ref):
    o = o_ref[...]
    y = (o - mu_ref[...]) * is_ref[...] * g_ref[...] + be_ref[...]
    y = jnp.maximum(y, 0.0)
    h2 = lax.dot_general(y, w2_ref[...], (((1,), (0,)), ((), ())),
                         preferred_element_type=jnp.float32, precision=HIGH)
    h_ref[...] = h2
    as_ref[0, 0, :] = jnp.sum(h2 * avs_ref[...], axis=1)
    ad_ref[0, 0, :] = jnp.sum(h2 * avd_ref[...], axis=1)


def _tc2b(o1, mu, istd, gamma, beta, W2, att_src, att_dst):
    return pl.pallas_call(
        _tc2b_body,
        grid=(RB,),
        in_specs=[
            pl.BlockSpec((RBS, DH), lambda i: (i, 0)),
            pl.BlockSpec((1, DH), lambda i: (0, 0)),
            pl.BlockSpec((1, DH), lambda i: (0, 0)),
            pl.BlockSpec((1, DH), lambda i: (0, 0)),
            pl.BlockSpec((1, DH), lambda i: (0, 0)),
            pl.BlockSpec((DH, DH), lambda i: (0, 0)),
            pl.BlockSpec((1, DH), lambda i: (0, 0)),
            pl.BlockSpec((1, DH), lambda i: (0, 0)),
        ],
        out_specs=[
            pl.BlockSpec((RBS, DH), lambda i: (i, 0)),
            pl.BlockSpec((1, 1, RBS), lambda i: (i, 0, 0)),
            pl.BlockSpec((1, 1, RBS), lambda i: (i, 0, 0)),
        ],
        out_shape=[
            jax.ShapeDtypeStruct((NP, DH), jnp.float32),
            jax.ShapeDtypeStruct((RB, 1, RBS), jnp.float32),
            jax.ShapeDtypeStruct((RB, 1, RBS), jnp.float32),
        ],
    )(o1, mu, istd, gamma, beta, W2, att_src, att_dst)


def _tc3_body(p_ref, b_ref, nb_ref, claim_ref, wc1_ref, wc2_ref, bc_ref,
              sum_ref, cnt_ref, out_ref):
    i = pl.program_id(0)
    o = p_ref[0] + p_ref[1] + b_ref[...]
    h = jnp.maximum(o, 0.0)
    nb = nb_ref[0, 0, :].reshape(RBS, 1)
    gid = lax.broadcasted_iota(jnp.int32, (RBS, BG), 1)
    onehot = (nb == gid).astype(jnp.float32)

    @pl.when(i == 0)
    def _init():
        sum_ref[...] = jnp.zeros_like(sum_ref)
        cnt_ref[...] = jnp.zeros_like(cnt_ref)

    sum_ref[...] += lax.dot_general(onehot, h, (((0,), (0,)), ((), ())),
                                    preferred_element_type=jnp.float32,
                                    precision=HIGH)
    ones = jnp.ones((RBS, DH), jnp.float32)
    cnt_ref[...] += lax.dot_general(onehot, ones, (((0,), (0,)), ((), ())),
                                    preferred_element_type=jnp.float32,
                                    precision=HIGH)

    @pl.when(i == RB - 1)
    def _final():
        pooled = sum_ref[...] / jnp.maximum(cnt_ref[...], 1.0)
        r = lax.dot_general(pooled, wc1_ref[...], (((1,), (0,)), ((), ())),
                            preferred_element_type=jnp.float32, precision=HIGH)
        r += lax.dot_general(claim_ref[...], wc2_ref[...],
                             (((1,), (0,)), ((), ())),
                             preferred_element_type=jnp.float32, precision=HIGH)
        out_ref[...] = r + bc_ref[...]


def _tc3(parts, b2, nb3, claim, Wc1, Wc2, bc):
    return pl.pallas_call(
        _tc3_body,
        grid=(RB,),
        in_specs=[
            pl.BlockSpec((2, RBS, DH), lambda i: (0, i, 0)),
            pl.BlockSpec((1, DH), lambda i: (0, 0)),
            pl.BlockSpec((1, 1, RBS), lambda i: (i, 0, 0)),
            pl.BlockSpec((BG, DIN), lambda i: (0, 0)),
            pl.BlockSpec((DH, 1), lambda i: (0, 0)),
            pl.BlockSpec((DIN, 1), lambda i: (0, 0)),
            pl.BlockSpec((1, 1), lambda i: (0, 0)),
        ],
        out_specs=[
            pl.BlockSpec((BG, DH), lambda i: (0, 0)),
            pl.BlockSpec((BG, DH), lambda i: (0, 0)),
            pl.BlockSpec((BG, 1), lambda i: (0, 0)),
        ],
        out_shape=[
            jax.ShapeDtypeStruct((BG, DH), jnp.float32),
            jax.ShapeDtypeStruct((BG, DH), jnp.float32),
            jax.ShapeDtypeStruct((BG, 1), jnp.float32),
        ],
    )(parts, b2, nb3, claim, Wc1, Wc2, bc)


def kernel(claim_embeddings, x, edge_index, node_batch,
           W1, att_src1, att_dst1, b1,
           W2, att_src2, att_dst2, b2,
           bn_gamma, bn_beta, Wc, bc):
    loop = jnp.arange(N, dtype=jnp.int32)
    padi = jnp.full((EP - E,), N, jnp.int32)
    src = jnp.concatenate([edge_index[0], loop, padi]).reshape(NW, NCHUNK, CH)
    dst = jnp.concatenate([edge_index[1], loop, padi]).reshape(NW, NCHUNK, CH)
    x_p = jnp.pad(x, ((0, NP - N), (0, 0)))
    nb3 = jnp.pad(node_batch.astype(jnp.int32), (0, NP - N),
                  constant_values=BG).reshape(RB, 1, RBS)
    avs1 = att_src1.reshape(1, DH)
    avd1 = att_dst1.reshape(1, DH)
    avs2 = att_src2.reshape(1, DH)
    avd2 = att_dst2.reshape(1, DH)

    h1, a_s, a_d = _tc1(x_p, nb3, claim_embeddings, W1, avs1, avd1)
    ee1, dp1 = _sca(src, dst, a_s.reshape(NP), a_d.reshape(NP))
    parts1 = _scb(src, dst, ee1, dp1, h1)

    o1, ps, pq = _tc2a(parts1, b1.reshape(1, DH))
    s = jnp.sum(ps.reshape(RB, DH), axis=0)
    q = jnp.sum(pq.reshape(RB, DH), axis=0)
    mu = s / N
    var = q / N - mu * mu
    istd = 1.0 / jnp.sqrt(var + 1e-5)
    h2, a_s2, a_d2 = _tc2b(o1, mu.reshape(1, DH), istd.reshape(1, DH),
                           bn_gamma.reshape(1, DH), bn_beta.reshape(1, DH),
                           W2, avs2, avd2)

    ee2, dp2 = _sca(src, dst, a_s2.reshape(NP), a_d2.reshape(NP))
    parts2 = _scb(src, dst, ee2, dp2, h2)

    _, _, out = _tc3(parts2, b2.reshape(1, DH), nb3, claim_embeddings,
                     Wc[:DH], Wc[DH:], bc.reshape(1, 1))
    return out.reshape(BG)


# speedup vs baseline: 19.9096x; 1.0144x over previous
import functools
import jax
import jax.numpy as jnp
from jax import lax
from jax.experimental import pallas as pl
from jax.experimental.pallas import tpu as pltpu
from jax.experimental.pallas import tpu_sc as plsc

N = 10000
NP = 10240
BG = 32
DIN = 768
DH = 128
E_RAW = 320000
E = E_RAW + N
NC, NS, L = 2, 16, 16
NW = NC * NS
CH = 128
NCHUNK = 81
ET = NCHUNK * CH
EP = NW * ET
STRIP = NP // NS
RB = 16
RBS = NP // RB
HIGH = lax.Precision.HIGHEST


def _tc1_body(x_ref, nb_ref, claim_ref, w1_ref, avs_ref, avd_ref,
              h_ref, as_ref, ad_ref):
    x = x_ref[...]
    nb = nb_ref[0, 0, :].reshape(RBS, 1)
    gid = lax.broadcasted_iota(jnp.int32, (RBS, BG), 1)
    onehot = (nb == gid).astype(jnp.float32)
    claim = claim_ref[...]
    ce = lax.dot_general(onehot, claim, (((1,), (0,)), ((), ())),
                         preferred_element_type=jnp.float32, precision=HIGH)
    dot = jnp.sum(ce * x, axis=1)
    na = jnp.sqrt(jnp.sum(ce * ce, axis=1))
    nx = jnp.sqrt(jnp.sum(x * x, axis=1))
    rel = dot / jnp.maximum(na * nx, 1e-8)
    h0 = x * rel[:, None]
    h1 = lax.dot_general(h0, w1_ref[...], (((1,), (0,)), ((), ())),
                         preferred_element_type=jnp.float32, precision=HIGH)
    h_ref[...] = h1
    as_ref[0, 0, :] = jnp.sum(h1 * avs_ref[...], axis=1)
    ad_ref[0, 0, :] = jnp.sum(h1 * avd_ref[...], axis=1)


def _tc1(x_p, nb3, claim, W1, att_src, att_dst):
    return pl.pallas_call(
        _tc1_body,
        grid=(RB,),
        in_specs=[
            pl.BlockSpec((RBS, DIN), lambda i: (i, 0)),
            pl.BlockSpec((1, 1, RBS), lambda i: (i, 0, 0)),
            pl.BlockSpec((BG, DIN), lambda i: (0, 0)),
            pl.BlockSpec((DIN, DH), lambda i: (0, 0)),
            pl.BlockSpec((1, DH), lambda i: (0, 0)),
            pl.BlockSpec((1, DH), lambda i: (0, 0)),
        ],
        out_specs=[
            pl.BlockSpec((RBS, DH), lambda i: (i, 0)),
            pl.BlockSpec((1, 1, RBS), lambda i: (i, 0, 0)),
            pl.BlockSpec((1, 1, RBS), lambda i: (i, 0, 0)),
        ],
        out_shape=[
            jax.ShapeDtypeStruct((NP, DH), jnp.float32),
            jax.ShapeDtypeStruct((RB, 1, RBS), jnp.float32),
            jax.ShapeDtypeStruct((RB, 1, RBS), jnp.float32),
        ],
    )(x_p, nb3, claim, W1, att_src, att_dst)


def _sca_body(src_h, dst_h, as_h, ad_h, ee_h, dpart_h,
              asrc_v, adst_v, srcv, dstv, eev, zbuf, denom_sp):
    cid = lax.axis_index("c")
    sid = lax.axis_index("s")
    wid = sid * NC + cid
    pltpu.sync_copy(as_h, asrc_v)
    pltpu.sync_copy(ad_h, adst_v)
    pltpu.sync_copy(src_h.at[wid], srcv)
    pltpu.sync_copy(dst_h.at[wid], dstv)
    z = jnp.zeros((L,), jnp.float32)

    def zb(i, carry):
        zbuf[pl.ds(i * L, L)] = z
        return carry
    lax.fori_loop(0, STRIP // L, zb, 0)
    pltpu.sync_copy(zbuf, denom_sp.at[pl.ds(sid * STRIP, STRIP)])
    plsc.subcore_barrier()

    def body(c, carry):
        for j in range(CH // L):
            s16 = srcv[c, pl.ds(j * L, L)]
            d16 = dstv[c, pl.ds(j * L, L)]
            t = plsc.load_gather(asrc_v, [s16]) + plsc.load_gather(adst_v, [d16])
            e = jnp.maximum(t, 0.2 * t)
            eev[c, pl.ds(j * L, L)] = jnp.exp(e)
        pltpu.sync_copy(eev.at[c], denom_sp.at[dstv.at[c]], add=True)
        return carry
    lax.fori_loop(0, NCHUNK, body, 0)
    pltpu.sync_copy(eev, ee_h.at[wid])
    plsc.subcore_barrier()
    pltpu.sync_copy(denom_sp.at[pl.ds(sid * STRIP, STRIP)],
                    dpart_h.at[cid, pl.ds(sid * STRIP, STRIP)])


def _sca(src2, dst2, a_src, a_dst):
    mesh = plsc.VectorSubcoreMesh(core_axis_name="c", subcore_axis_name="s", num_cores=NC, num_subcores=NS)
    f = pl.kernel(
        _sca_body,
        out_type=[
            jax.ShapeDtypeStruct((NW, NCHUNK, CH), jnp.float32),
            jax.ShapeDtypeStruct((NC, NP), jnp.float32),
        ],
        mesh=mesh,
        compiler_params=pltpu.CompilerParams(needs_layout_passes=False),
        scratch_types=[
            pltpu.VMEM((NP,), jnp.float32),
            pltpu.VMEM((NP,), jnp.float32),
            pltpu.VMEM((NCHUNK, CH), jnp.int32),
            pltpu.VMEM((NCHUNK, CH), jnp.int32),
            pltpu.VMEM((NCHUNK, CH), jnp.float32),
            pltpu.VMEM((STRIP,), jnp.float32),
            pltpu.VMEM_SHARED((NP,), jnp.float32),
        ],
    )
    return f(src2, dst2, a_src, a_dst)


C2 = 64
NCH2 = ET // C2


def _scb_body(src_h, dst_h, ee_h, dp_h, h_h, part_h,
              inv_v, d1s, rowsA, rowsB, iA, iB, dA, dB, eA, eB,
              gsA, gsB, ssA, ssB, out_sp):
    cid = lax.axis_index("c")
    sid = lax.axis_index("s")
    wid = sid * NC + cid
    pltpu.sync_copy(dp_h.at[0], inv_v)

    def invt(t, carry):
        pltpu.sync_copy(dp_h.at[1].at[pl.ds(t * STRIP, STRIP)], d1s)

        def invb(k, c2):
            s = pl.ds(t * STRIP + k * L, L)
            inv_v[s] = 1.0 / (inv_v[s] + d1s[pl.ds(k * L, L)] + 1e-16)
            return c2
        lax.fori_loop(0, STRIP // L, invb, 0)
        return carry
    lax.fori_loop(0, NS, invt, 0)

    z = jnp.zeros((L,), jnp.float32)

    def zrow(i, carry):
        for j in range(DH // L):
            rowsA[i, pl.ds(j * L, L)] = z
        return carry
    lax.fori_loop(0, C2, zrow, 0)

    def zstrip(k, carry):
        pltpu.sync_copy(rowsA, out_sp.at[pl.ds(sid * STRIP + k * C2, C2)])
        return carry
    lax.fori_loop(0, STRIP // C2, zstrip, 0)
    plsc.subcore_barrier()

    def fetch(c, ibuf, dbuf, ebuf):
        pltpu.sync_copy(src_h.at[wid].at[c], ibuf)
        pltpu.sync_copy(dst_h.at[wid].at[c], dbuf)
        pltpu.sync_copy(ee_h.at[wid].at[c], ebuf)

    def compute(dbuf, ebuf, rbuf):
        def scale_j(j, carry):
            d16 = dbuf[pl.ds(j * L, L)]
            a16 = ebuf[pl.ds(j * L, L)] * plsc.load_gather(inv_v, [d16])
            for k in range(L):
                s = a16[k]
                i = j * L + k
                for jj in range(DH // L):
                    sl = pl.ds(jj * L, L)
                    rbuf[i, sl] = rbuf[i, sl] * s
            return carry
        lax.fori_loop(0, C2 // L, scale_j, 0)

    fetch(0, iA, dA, eA)
    pltpu.async_copy(h_h.at[iA], rowsA, gsA)

    def pair(g, carry):
        cB = 2 * g + 1
        @pl.when(g > 0)
        def _wb():
            pltpu.make_async_copy(rowsB, out_sp.at[dB], ssB).wait()
        fetch(cB, iB, dB, eB)
        pltpu.async_copy(h_h.at[iB], rowsB, gsB)
        pltpu.make_async_copy(h_h.at[iA], rowsA, gsA).wait()
        compute(dA, eA, rowsA)
        pltpu.async_copy(rowsA, out_sp.at[dA], ssA, add=True)

        @pl.when(g < NCH2 // 2 - 1)
        def _na():
            pltpu.make_async_copy(rowsA, out_sp.at[dA], ssA).wait()
            fetch(2 * g + 2, iA, dA, eA)
            pltpu.async_copy(h_h.at[iA], rowsA, gsA)
        pltpu.make_async_copy(h_h.at[iB], rowsB, gsB).wait()
        compute(dB, eB, rowsB)
        pltpu.async_copy(rowsB, out_sp.at[dB], ssB, add=True)
        return carry
    lax.fori_loop(0, NCH2 // 2, pair, 0)
    pltpu.make_async_copy(rowsA, out_sp.at[dA], ssA).wait()
    pltpu.make_async_copy(rowsB, out_sp.at[dB], ssB).wait()
    plsc.subcore_barrier()
    pltpu.sync_copy(out_sp.at[pl.ds(sid * STRIP, STRIP)],
                    part_h.at[cid, pl.ds(sid * STRIP, STRIP)])


def _scb(src2, dst2, ee, dparts, h):
    mesh = plsc.VectorSubcoreMesh(core_axis_name="c", subcore_axis_name="s", num_cores=NC, num_subcores=NS)
    f = pl.kernel(
        _scb_body,
        out_type=jax.ShapeDtypeStruct((NC, NP, DH), jnp.float32),
        mesh=mesh,
        compiler_params=pltpu.CompilerParams(needs_layout_passes=False),
        scratch_types=[
            pltpu.VMEM((NP,), jnp.float32),
            pltpu.VMEM((STRIP,), jnp.float32),
            pltpu.VMEM((C2, DH), jnp.float32),
            pltpu.VMEM((C2, DH), jnp.float32),
            pltpu.VMEM((C2,), jnp.int32),
            pltpu.VMEM((C2,), jnp.int32),
            pltpu.VMEM((C2,), jnp.int32),
            pltpu.VMEM((C2,), jnp.int32),
            pltpu.VMEM((C2,), jnp.float32),
            pltpu.VMEM((C2,), jnp.float32),
            pltpu.SemaphoreType.DMA,
            pltpu.SemaphoreType.DMA,
            pltpu.SemaphoreType.DMA,
            pltpu.SemaphoreType.DMA,
            pltpu.VMEM_SHARED((NP, DH), jnp.float32),
        ],
    )
    return f(src2, dst2, ee, dparts, h)


def _tc2a_body(p_ref, b_ref, o_ref, s_ref, q_ref):
    i = pl.program_id(0)
    o = p_ref[0] + p_ref[1] + b_ref[...]
    o_ref[...] = o
    rid = lax.broadcasted_iota(jnp.int32, (RBS, 1), 0) + i * RBS
    m = (rid < N).astype(jnp.float32)
    om = o * m
    s_ref[0, :, :] = jnp.sum(om, axis=0, keepdims=True)
    q_ref[0, :, :] = jnp.sum(om * o, axis=0, keepdims=True)


def _tc2a(parts, b1):
    return pl.pallas_call(
        _tc2a_body,
        grid=(RB,),
        in_specs=[
            pl.BlockSpec((2, RBS, DH), lambda i: (0, i, 0)),
            pl.BlockSpec((1, DH), lambda i: (0, 0)),
        ],
        out_specs=[
            pl.BlockSpec((RBS, DH), lambda i: (i, 0)),
            pl.BlockSpec((1, 1, DH), lambda i: (i, 0, 0)),
            pl.BlockSpec((1, 1, DH), lambda i: (i, 0, 0)),
        ],
        out_shape=[
            jax.ShapeDtypeStruct((NP, DH), jnp.float32),
            jax.ShapeDtypeStruct((RB, 1, DH), jnp.float32),
            jax.ShapeDtypeStruct((RB, 1, DH), jnp.float32),
        ],
    )(parts, b1)


def _tc2b_body(o_ref, mu_ref, is_ref, g_ref, be_ref, w2_ref, avs_ref, avd_ref,
               h_ref, as_ref, ad_ref):
    o = o_ref[...]
    y = (o - mu_ref[...]) * is_ref[...] * g_ref[...] + be_ref[...]
    y = jnp.maximum(y, 0.0)
    h2 = lax.dot_general(y, w2_ref[...], (((1,), (0,)), ((), ())),
                         preferred_element_type=jnp.float32, precision=HIGH)
    h_ref[...] = h2
    as_ref[0, 0, :] = jnp.sum(h2 * avs_ref[...], axis=1)
    ad_ref[0, 0, :] = jnp.sum(h2 * avd_ref[...], axis=1)


def _tc2b(o1, mu, istd, gamma, beta, W2, att_src, att_dst):
    return pl.pallas_call(
        _tc2b_body,
        grid=(RB,),
        in_specs=[
            pl.BlockSpec((RBS, DH), lambda i: (i, 0)),
            pl.BlockSpec((1, DH), lambda i: (0, 0)),
            pl.BlockSpec((1, DH), lambda i: (0, 0)),
            pl.BlockSpec((1, DH), lambda i: (0, 0)),
            pl.BlockSpec((1, DH), lambda i: (0, 0)),
            pl.BlockSpec((DH, DH), lambda i: (0, 0)),
            pl.BlockSpec((1, DH), lambda i: (0, 0)),
            pl.BlockSpec((1, DH), lambda i: (0, 0)),
        ],
        out_specs=[
            pl.BlockSpec((RBS, DH), lambda i: (i, 0)),
            pl.BlockSpec((1, 1, RBS), lambda i: (i, 0, 0)),
            pl.BlockSpec((1, 1, RBS), lambda i: (i, 0, 0)),
        ],
        out_shape=[
            jax.ShapeDtypeStruct((NP, DH), jnp.float32),
            jax.ShapeDtypeStruct((RB, 1, RBS), jnp.float32),
            jax.ShapeDtypeStruct((RB, 1, RBS), jnp.float32),
        ],
    )(o1, mu, istd, gamma, beta, W2, att_src, att_dst)


def _tc3_body(p_ref, b_ref, nb_ref, claim_ref, wc1_ref, wc2_ref, bc_ref,
              sum_ref, cnt_ref, out_ref):
    i = pl.program_id(0)
    o = p_ref[0] + p_ref[1] + b_ref[...]
    h = jnp.maximum(o, 0.0)
    nb = nb_ref[0, 0, :].reshape(RBS, 1)
    gid = lax.broadcasted_iota(jnp.int32, (RBS, BG), 1)
    onehot = (nb == gid).astype(jnp.float32)

    @pl.when(i == 0)
    def _init():
        sum_ref[...] = jnp.zeros_like(sum_ref)
        cnt_ref[...] = jnp.zeros_like(cnt_ref)

    sum_ref[...] += lax.dot_general(onehot, h, (((0,), (0,)), ((), ())),
                                    preferred_element_type=jnp.float32,
                                    precision=HIGH)
    ones = jnp.ones((RBS, DH), jnp.float32)
    cnt_ref[...] += lax.dot_general(onehot, ones, (((0,), (0,)), ((), ())),
                                    preferred_element_type=jnp.float32,
                                    precision=HIGH)

    @pl.when(i == RB - 1)
    def _final():
        pooled = sum_ref[...] / jnp.maximum(cnt_ref[...], 1.0)
        r = lax.dot_general(pooled, wc1_ref[...], (((1,), (0,)), ((), ())),
                            preferred_element_type=jnp.float32, precision=HIGH)
        r += lax.dot_general(claim_ref[...], wc2_ref[...],
                             (((1,), (0,)), ((), ())),
                             preferred_element_type=jnp.float32, precision=HIGH)
        out_ref[...] = r + bc_ref[...]


def _tc3(parts, b2, nb3, claim, Wc1, Wc2, bc):
    return pl.pallas_call(
        _tc3_body,
        grid=(RB,),
        in_specs=[
            pl.BlockSpec((2, RBS, DH), lambda i: (0, i, 0)),
            pl.BlockSpec((1, DH), lambda i: (0, 0)),
            pl.BlockSpec((1, 1, RBS), lambda i: (i, 0, 0)),
            pl.BlockSpec((BG, DIN), lambda i: (0, 0)),
            pl.BlockSpec((DH, 1), lambda i: (0, 0)),
            pl.BlockSpec((DIN, 1), lambda i: (0, 0)),
            pl.BlockSpec((1, 1), lambda i: (0, 0)),
        ],
        out_specs=[
            pl.BlockSpec((BG, DH), lambda i: (0, 0)),
            pl.BlockSpec((BG, DH), lambda i: (0, 0)),
            pl.BlockSpec((BG, 1), lambda i: (0, 0)),
        ],
        out_shape=[
            jax.ShapeDtypeStruct((BG, DH), jnp.float32),
            jax.ShapeDtypeStruct((BG, DH), jnp.float32),
            jax.ShapeDtypeStruct((BG, 1), jnp.float32),
        ],
    )(parts, b2, nb3, claim, Wc1, Wc2, bc)


def kernel(claim_embeddings, x, edge_index, node_batch,
           W1, att_src1, att_dst1, b1,
           W2, att_src2, att_dst2, b2,
           bn_gamma, bn_beta, Wc, bc):
    loop = jnp.arange(N, dtype=jnp.int32)
    padi = jnp.full((EP - E,), N, jnp.int32)
    src = jnp.concatenate([edge_index[0], loop, padi]).reshape(NW, NCHUNK, CH)
    dst = jnp.concatenate([edge_index[1], loop, padi]).reshape(NW, NCHUNK, CH)
    x_p = jnp.pad(x, ((0, NP - N), (0, 0)))
    nb3 = jnp.pad(node_batch.astype(jnp.int32), (0, NP - N),
                  constant_values=BG).reshape(RB, 1, RBS)
    avs1 = att_src1.reshape(1, DH)
    avd1 = att_dst1.reshape(1, DH)
    avs2 = att_src2.reshape(1, DH)
    avd2 = att_dst2.reshape(1, DH)

    src64 = src.reshape(NW, NCH2, C2)
    dst64 = dst.reshape(NW, NCH2, C2)

    h1, a_s, a_d = _tc1(x_p, nb3, claim_embeddings, W1, avs1, avd1)
    ee1, dp1 = _sca(src, dst, a_s.reshape(NP), a_d.reshape(NP))
    parts1 = _scb(src64, dst64, ee1.reshape(NW, NCH2, C2), dp1, h1)

    o1, ps, pq = _tc2a(parts1, b1.reshape(1, DH))
    s = jnp.sum(ps.reshape(RB, DH), axis=0)
    q = jnp.sum(pq.reshape(RB, DH), axis=0)
    mu = s / N
    var = q / N - mu * mu
    istd = 1.0 / jnp.sqrt(var + 1e-5)
    h2, a_s2, a_d2 = _tc2b(o1, mu.reshape(1, DH), istd.reshape(1, DH),
                           bn_gamma.reshape(1, DH), bn_beta.reshape(1, DH),
                           W2, avs2, avd2)

    ee2, dp2 = _sca(src, dst, a_s2.reshape(NP), a_d2.reshape(NP))
    parts2 = _scb(src64, dst64, ee2.reshape(NW, NCH2, C2), dp2, h2)

    _, _, out = _tc3(parts2, b2.reshape(1, DH), nb3, claim_embeddings,
                     Wc[:DH], Wc[DH:], bc.reshape(1, 1))
    return out.reshape(BG)


# speedup vs baseline: 23.2875x; 1.1697x over previous
import functools
import jax
import jax.numpy as jnp
from jax import lax
from jax.experimental import pallas as pl
from jax.experimental.pallas import tpu as pltpu
from jax.experimental.pallas import tpu_sc as plsc

N = 10000
NP = 10240
BG = 32
DIN = 768
DH = 128
E_RAW = 320000
E = E_RAW + N
NC, NS, L = 2, 16, 16
NW = NC * NS
CH = 128
NCHUNK = 81
ET = NCHUNK * CH
EP = NW * ET
STRIP = NP // NS
RB = 16
RBS = NP // RB
HIGH = lax.Precision.HIGHEST


def _tc1_body(x_ref, nb_ref, claim_ref, w1_ref, avs_ref, avd_ref,
              h_ref, as_ref, ad_ref):
    x = x_ref[...]
    nb = nb_ref[0, 0, :].reshape(RBS, 1)
    gid = lax.broadcasted_iota(jnp.int32, (RBS, BG), 1)
    onehot = (nb == gid).astype(jnp.float32)
    claim = claim_ref[...]
    ce = lax.dot_general(onehot, claim, (((1,), (0,)), ((), ())),
                         preferred_element_type=jnp.float32, precision=HIGH)
    dot = jnp.sum(ce * x, axis=1)
    na = jnp.sqrt(jnp.sum(ce * ce, axis=1))
    nx = jnp.sqrt(jnp.sum(x * x, axis=1))
    rel = dot / jnp.maximum(na * nx, 1e-8)
    h0 = x * rel[:, None]
    h1 = lax.dot_general(h0, w1_ref[...], (((1,), (0,)), ((), ())),
                         preferred_element_type=jnp.float32, precision=HIGH)
    h_ref[...] = h1
    as_ref[0, 0, :] = jnp.sum(h1 * avs_ref[...], axis=1)
    ad_ref[0, 0, :] = jnp.sum(h1 * avd_ref[...], axis=1)


def _tc1(x_p, nb3, claim, W1, att_src, att_dst):
    return pl.pallas_call(
        _tc1_body,
        grid=(RB,),
        in_specs=[
            pl.BlockSpec((RBS, DIN), lambda i: (i, 0)),
            pl.BlockSpec((1, 1, RBS), lambda i: (i, 0, 0)),
            pl.BlockSpec((BG, DIN), lambda i: (0, 0)),
            pl.BlockSpec((DIN, DH), lambda i: (0, 0)),
            pl.BlockSpec((1, DH), lambda i: (0, 0)),
            pl.BlockSpec((1, DH), lambda i: (0, 0)),
        ],
        out_specs=[
            pl.BlockSpec((RBS, DH), lambda i: (i, 0)),
            pl.BlockSpec((1, 1, RBS), lambda i: (i, 0, 0)),
            pl.BlockSpec((1, 1, RBS), lambda i: (i, 0, 0)),
        ],
        out_shape=[
            jax.ShapeDtypeStruct((NP, DH), jnp.float32),
            jax.ShapeDtypeStruct((RB, 1, RBS), jnp.float32),
            jax.ShapeDtypeStruct((RB, 1, RBS), jnp.float32),
        ],
    )(x_p, nb3, claim, W1, att_src, att_dst)


def _sca_body(src_h, dst_h, as_h, ad_h, ee_h, dpart_h,
              asrc_v, adst_v, srcv, dstv, eev, zbuf, denom_sp):
    cid = lax.axis_index("c")
    sid = lax.axis_index("s")
    wid = sid * NC + cid
    pltpu.sync_copy(as_h, asrc_v)
    pltpu.sync_copy(ad_h, adst_v)
    pltpu.sync_copy(src_h.at[wid], srcv)
    pltpu.sync_copy(dst_h.at[wid], dstv)
    z = jnp.zeros((L,), jnp.float32)

    def zb(i, carry):
        zbuf[pl.ds(i * L, L)] = z
        return carry
    lax.fori_loop(0, STRIP // L, zb, 0)
    pltpu.sync_copy(zbuf, denom_sp.at[pl.ds(sid * STRIP, STRIP)])
    plsc.subcore_barrier()

    def body(c, carry):
        for j in range(CH // L):
            s16 = srcv[c, pl.ds(j * L, L)]
            d16 = dstv[c, pl.ds(j * L, L)]
            t = plsc.load_gather(asrc_v, [s16]) + plsc.load_gather(adst_v, [d16])
            e = jnp.maximum(t, 0.2 * t)
            eev[c, pl.ds(j * L, L)] = jnp.exp(e)
        pltpu.sync_copy(eev.at[c], denom_sp.at[dstv.at[c]], add=True)
        return carry
    lax.fori_loop(0, NCHUNK, body, 0)
    pltpu.sync_copy(eev, ee_h.at[wid])
    plsc.subcore_barrier()
    pltpu.sync_copy(denom_sp.at[pl.ds(sid * STRIP, STRIP)],
                    dpart_h.at[cid, pl.ds(sid * STRIP, STRIP)])


def _sca(src2, dst2, a_src, a_dst):
    mesh = plsc.VectorSubcoreMesh(core_axis_name="c", subcore_axis_name="s", num_cores=NC, num_subcores=NS)
    f = pl.kernel(
        _sca_body,
        out_type=[
            jax.ShapeDtypeStruct((NW, NCHUNK, CH), jnp.float32),
            jax.ShapeDtypeStruct((NC, NP), jnp.float32),
        ],
        mesh=mesh,
        compiler_params=pltpu.CompilerParams(needs_layout_passes=False),
        scratch_types=[
            pltpu.VMEM((NP,), jnp.float32),
            pltpu.VMEM((NP,), jnp.float32),
            pltpu.VMEM((NCHUNK, CH), jnp.int32),
            pltpu.VMEM((NCHUNK, CH), jnp.int32),
            pltpu.VMEM((NCHUNK, CH), jnp.float32),
            pltpu.VMEM((STRIP,), jnp.float32),
            pltpu.VMEM_SHARED((NP,), jnp.float32),
        ],
    )
    return f(src2, dst2, a_src, a_dst)


C2 = 64
NCH2 = ET // C2


def _scb_body(src_h, dst_h, ee_h, dp_h, h_h, part_h,
              inv_v, d1s, rowsA, rowsB, iA, iB, dA, dB, eA, eB,
              gsA, gsB, ssA, ssB, fsA, fsB, out_sp):
    cid = lax.axis_index("c")
    sid = lax.axis_index("s")
    wid = sid * NC + cid
    pltpu.sync_copy(dp_h.at[0], inv_v)

    def invt(t, carry):
        pltpu.sync_copy(dp_h.at[1].at[pl.ds(t * STRIP, STRIP)], d1s)

        def invb(k, c2):
            s = pl.ds(t * STRIP + k * L, L)
            inv_v[s] = 1.0 / (inv_v[s] + d1s[pl.ds(k * L, L)] + 1e-16)
            return c2
        lax.fori_loop(0, STRIP // L, invb, 0)
        return carry
    lax.fori_loop(0, NS, invt, 0)

    z = jnp.zeros((L,), jnp.float32)

    def zrow(i, carry):
        for j in range(DH // L):
            rowsA[i, pl.ds(j * L, L)] = z
        return carry
    lax.fori_loop(0, C2, zrow, 0)

    def zstrip(k, carry):
        pltpu.sync_copy(rowsA, out_sp.at[pl.ds(sid * STRIP + k * C2, C2)])
        return carry
    lax.fori_loop(0, STRIP // C2, zstrip, 0)
    plsc.subcore_barrier()

    def fetch(c, ibuf, dbuf, ebuf, fsem):
        pltpu.async_copy(src_h.at[wid].at[c], ibuf, fsem)
        pltpu.async_copy(dst_h.at[wid].at[c], dbuf, fsem)
        pltpu.async_copy(ee_h.at[wid].at[c], ebuf, fsem)

    def fetch_wait(c, ibuf, dbuf, ebuf, fsem):
        pltpu.make_async_copy(src_h.at[wid].at[c], ibuf, fsem).wait()
        pltpu.make_async_copy(dst_h.at[wid].at[c], dbuf, fsem).wait()
        pltpu.make_async_copy(ee_h.at[wid].at[c], ebuf, fsem).wait()

    def compute(dbuf, ebuf, rbuf):
        def scale_j(j, carry):
            d16 = dbuf[pl.ds(j * L, L)]
            a16 = ebuf[pl.ds(j * L, L)] * plsc.load_gather(inv_v, [d16])
            for k in range(L):
                s = a16[k]
                i = j * L + k
                for jj in range(DH // L):
                    sl = pl.ds(jj * L, L)
                    rbuf[i, sl] = rbuf[i, sl] * s
            return carry
        lax.fori_loop(0, C2 // L, scale_j, 0)

    fetch(0, iA, dA, eA, fsA)
    fetch_wait(0, iA, dA, eA, fsA)
    pltpu.async_copy(h_h.at[iA], rowsA, gsA)

    def pair(g, carry):
        cB = 2 * g + 1
        @pl.when(g > 0)
        def _wb():
            pltpu.make_async_copy(rowsB, out_sp.at[dB], ssB).wait()
        fetch(cB, iB, dB, eB, fsB)
        pltpu.make_async_copy(h_h.at[iA], rowsA, gsA).wait()
        compute(dA, eA, rowsA)
        pltpu.async_copy(rowsA, out_sp.at[dA], ssA, add=True)
        fetch_wait(cB, iB, dB, eB, fsB)
        pltpu.async_copy(h_h.at[iB], rowsB, gsB)

        @pl.when(g < NCH2 // 2 - 1)
        def _na():
            pltpu.make_async_copy(rowsA, out_sp.at[dA], ssA).wait()
            fetch(2 * g + 2, iA, dA, eA, fsA)
        pltpu.make_async_copy(h_h.at[iB], rowsB, gsB).wait()
        compute(dB, eB, rowsB)
        pltpu.async_copy(rowsB, out_sp.at[dB], ssB, add=True)

        @pl.when(g < NCH2 // 2 - 1)
        def _ga():
            fetch_wait(2 * g + 2, iA, dA, eA, fsA)
            pltpu.async_copy(h_h.at[iA], rowsA, gsA)
        return carry
    lax.fori_loop(0, NCH2 // 2, pair, 0)
    pltpu.make_async_copy(rowsA, out_sp.at[dA], ssA).wait()
    pltpu.make_async_copy(rowsB, out_sp.at[dB], ssB).wait()
    plsc.subcore_barrier()
    pltpu.sync_copy(out_sp.at[pl.ds(sid * STRIP, STRIP)],
                    part_h.at[cid, pl.ds(sid * STRIP, STRIP)])


def _scb(src2, dst2, ee, dparts, h):
    mesh = plsc.VectorSubcoreMesh(core_axis_name="c", subcore_axis_name="s", num_cores=NC, num_subcores=NS)
    f = pl.kernel(
        _scb_body,
        out_type=jax.ShapeDtypeStruct((NC, NP, DH), jnp.float32),
        mesh=mesh,
        compiler_params=pltpu.CompilerParams(needs_layout_passes=False),
        scratch_types=[
            pltpu.VMEM((NP,), jnp.float32),
            pltpu.VMEM((STRIP,), jnp.float32),
            pltpu.VMEM((C2, DH), jnp.float32),
            pltpu.VMEM((C2, DH), jnp.float32),
            pltpu.VMEM((C2,), jnp.int32),
            pltpu.VMEM((C2,), jnp.int32),
            pltpu.VMEM((C2,), jnp.int32),
            pltpu.VMEM((C2,), jnp.int32),
            pltpu.VMEM((C2,), jnp.float32),
            pltpu.VMEM((C2,), jnp.float32),
            pltpu.SemaphoreType.DMA,
            pltpu.SemaphoreType.DMA,
            pltpu.SemaphoreType.DMA,
            pltpu.SemaphoreType.DMA,
            pltpu.SemaphoreType.DMA,
            pltpu.SemaphoreType.DMA,
            pltpu.VMEM_SHARED((NP, DH), jnp.float32),
        ],
    )
    return f(src2, dst2, ee, dparts, h)


def _tc2a_body(p_ref, b_ref, o_ref, s_ref, q_ref):
    i = pl.program_id(0)
    o = p_ref[0] + p_ref[1] + b_ref[...]
    o_ref[...] = o
    rid = lax.broadcasted_iota(jnp.int32, (RBS, 1), 0) + i * RBS
    m = (rid < N).astype(jnp.float32)
    om = o * m
    s_ref[0, :, :] = jnp.sum(om, axis=0, keepdims=True)
    q_ref[0, :, :] = jnp.sum(om * o, axis=0, keepdims=True)


def _tc2a(parts, b1):
    return pl.pallas_call(
        _tc2a_body,
        grid=(RB,),
        in_specs=[
            pl.BlockSpec((2, RBS, DH), lambda i: (0, i, 0)),
            pl.BlockSpec((1, DH), lambda i: (0, 0)),
        ],
        out_specs=[
            pl.BlockSpec((RBS, DH), lambda i: (i, 0)),
            pl.BlockSpec((1, 1, DH), lambda i: (i, 0, 0)),
            pl.BlockSpec((1, 1, DH), lambda i: (i, 0, 0)),
        ],
        out_shape=[
            jax.ShapeDtypeStruct((NP, DH), jnp.float32),
            jax.ShapeDtypeStruct((RB, 1, DH), jnp.float32),
            jax.ShapeDtypeStruct((RB, 1, DH), jnp.float32),
        ],
    )(parts, b1)


def _tc2b_body(o_ref, mu_ref, is_ref, g_ref, be_ref, w2_ref, avs_ref, avd_ref,
               h_ref, as_ref, ad_ref):
    o = o_ref[...]
    y = (o - mu_ref[...]) * is_ref[...] * g_ref[...] + be_ref[...]
    y = jnp.maximum(y, 0.0)
    h2 = lax.dot_general(y, w2_ref[...], (((1,), (0,)), ((), ())),
                         preferred_element_type=jnp.float32, precision=HIGH)
    h_ref[...] = h2
    as_ref[0, 0, :] = jnp.sum(h2 * avs_ref[...], axis=1)
    ad_ref[0, 0, :] = jnp.sum(h2 * avd_ref[...], axis=1)


def _tc2b(o1, mu, istd, gamma, beta, W2, att_src, att_dst):
    return pl.pallas_call(
        _tc2b_body,
        grid=(RB,),
        in_specs=[
            pl.BlockSpec((RBS, DH), lambda i: (i, 0)),
            pl.BlockSpec((1, DH), lambda i: (0, 0)),
            pl.BlockSpec((1, DH), lambda i: (0, 0)),
            pl.BlockSpec((1, DH), lambda i: (0, 0)),
            pl.BlockSpec((1, DH), lambda i: (0, 0)),
            pl.BlockSpec((DH, DH), lambda i: (0, 0)),
            pl.BlockSpec((1, DH), lambda i: (0, 0)),
            pl.BlockSpec((1, DH), lambda i: (0, 0)),
        ],
        out_specs=[
            pl.BlockSpec((RBS, DH), lambda i: (i, 0)),
            pl.BlockSpec((1, 1, RBS), lambda i: (i, 0, 0)),
            pl.BlockSpec((1, 1, RBS), lambda i: (i, 0, 0)),
        ],
        out_shape=[
            jax.ShapeDtypeStruct((NP, DH), jnp.float32),
            jax.ShapeDtypeStruct((RB, 1, RBS), jnp.float32),
            jax.ShapeDtypeStruct((RB, 1, RBS), jnp.float32),
        ],
    )(o1, mu, istd, gamma, beta, W2, att_src, att_dst)


def _tc3_body(p_ref, b_ref, nb_ref, claim_ref, wc1_ref, wc2_ref, bc_ref,
              sum_ref, cnt_ref, out_ref):
    i = pl.program_id(0)
    o = p_ref[0] + p_ref[1] + b_ref[...]
    h = jnp.maximum(o, 0.0)
    nb = nb_ref[0, 0, :].reshape(RBS, 1)
    gid = lax.broadcasted_iota(jnp.int32, (RBS, BG), 1)
    onehot = (nb == gid).astype(jnp.float32)

    @pl.when(i == 0)
    def _init():
        sum_ref[...] = jnp.zeros_like(sum_ref)
        cnt_ref[...] = jnp.zeros_like(cnt_ref)

    sum_ref[...] += lax.dot_general(onehot, h, (((0,), (0,)), ((), ())),
                                    preferred_element_type=jnp.float32,
                                    precision=HIGH)
    ones = jnp.ones((RBS, DH), jnp.float32)
    cnt_ref[...] += lax.dot_general(onehot, ones, (((0,), (0,)), ((), ())),
                                    preferred_element_type=jnp.float32,
                                    precision=HIGH)

    @pl.when(i == RB - 1)
    def _final():
        pooled = sum_ref[...] / jnp.maximum(cnt_ref[...], 1.0)
        r = lax.dot_general(pooled, wc1_ref[...], (((1,), (0,)), ((), ())),
                            preferred_element_type=jnp.float32, precision=HIGH)
        r += lax.dot_general(claim_ref[...], wc2_ref[...],
                             (((1,), (0,)), ((), ())),
                             preferred_element_type=jnp.float32, precision=HIGH)
        out_ref[...] = r + bc_ref[...]


def _tc3(parts, b2, nb3, claim, Wc1, Wc2, bc):
    return pl.pallas_call(
        _tc3_body,
        grid=(RB,),
        in_specs=[
            pl.BlockSpec((2, RBS, DH), lambda i: (0, i, 0)),
            pl.BlockSpec((1, DH), lambda i: (0, 0)),
            pl.BlockSpec((1, 1, RBS), lambda i: (i, 0, 0)),
            pl.BlockSpec((BG, DIN), lambda i: (0, 0)),
            pl.BlockSpec((DH, 1), lambda i: (0, 0)),
            pl.BlockSpec((DIN, 1), lambda i: (0, 0)),
            pl.BlockSpec((1, 1), lambda i: (0, 0)),
        ],
        out_specs=[
            pl.BlockSpec((BG, DH), lambda i: (0, 0)),
            pl.BlockSpec((BG, DH), lambda i: (0, 0)),
            pl.BlockSpec((BG, 1), lambda i: (0, 0)),
        ],
        out_shape=[
            jax.ShapeDtypeStruct((BG, DH), jnp.float32),
            jax.ShapeDtypeStruct((BG, DH), jnp.float32),
            jax.ShapeDtypeStruct((BG, 1), jnp.float32),
        ],
    )(parts, b2, nb3, claim, Wc1, Wc2, bc)


def kernel(claim_embeddings, x, edge_index, node_batch,
           W1, att_src1, att_dst1, b1,
           W2, att_src2, att_dst2, b2,
           bn_gamma, bn_beta, Wc, bc):
    loop = jnp.arange(N, dtype=jnp.int32)
    padi = jnp.full((EP - E,), N, jnp.int32)
    src = jnp.concatenate([edge_index[0], loop, padi]).reshape(NW, NCHUNK, CH)
    dst = jnp.concatenate([edge_index[1], loop, padi]).reshape(NW, NCHUNK, CH)
    x_p = jnp.pad(x, ((0, NP - N), (0, 0)))
    nb3 = jnp.pad(node_batch.astype(jnp.int32), (0, NP - N),
                  constant_values=BG).reshape(RB, 1, RBS)
    avs1 = att_src1.reshape(1, DH)
    avd1 = att_dst1.reshape(1, DH)
    avs2 = att_src2.reshape(1, DH)
    avd2 = att_dst2.reshape(1, DH)

    src64 = src.reshape(NW, NCH2, C2)
    dst64 = dst.reshape(NW, NCH2, C2)

    h1, a_s, a_d = _tc1(x_p, nb3, claim_embeddings, W1, avs1, avd1)
    ee1, dp1 = _sca(src, dst, a_s.reshape(NP), a_d.reshape(NP))
    parts1 = _scb(src64, dst64, ee1.reshape(NW, NCH2, C2), dp1, h1)

    o1, ps, pq = _tc2a(parts1, b1.reshape(1, DH))
    s = jnp.sum(ps.reshape(RB, DH), axis=0)
    q = jnp.sum(pq.reshape(RB, DH), axis=0)
    mu = s / N
    var = q / N - mu * mu
    istd = 1.0 / jnp.sqrt(var + 1e-5)
    h2, a_s2, a_d2 = _tc2b(o1, mu.reshape(1, DH), istd.reshape(1, DH),
                           bn_gamma.reshape(1, DH), bn_beta.reshape(1, DH),
                           W2, avs2, avd2)

    ee2, dp2 = _sca(src, dst, a_s2.reshape(NP), a_d2.reshape(NP))
    parts2 = _scb(src64, dst64, ee2.reshape(NW, NCH2, C2), dp2, h2)

    _, _, out = _tc3(parts2, b2.reshape(1, DH), nb3, claim_embeddings,
                     Wc[:DH], Wc[DH:], bc.reshape(1, 1))
    return out.reshape(BG)


# speedup vs baseline: 25.5630x; 1.0977x over previous
import functools
import jax
import jax.numpy as jnp
from jax import lax
from jax.experimental import pallas as pl
from jax.experimental.pallas import tpu as pltpu
from jax.experimental.pallas import tpu_sc as plsc

N = 10000
NP = 10240
BG = 32
DIN = 768
DH = 128
E_RAW = 320000
E = E_RAW + N
NC, NS, L = 2, 16, 16
NW = NC * NS
CH = 128
NCHUNK = 81
ET = NCHUNK * CH
EP = NW * ET
STRIP = NP // NS
RB = 16
RBS = NP // RB
HIGH = lax.Precision.HIGHEST


def _tc1_body(x_ref, nb_ref, claim_ref, w1_ref, avs_ref, avd_ref,
              h_ref, as_ref, ad_ref):
    x = x_ref[...]
    nb = nb_ref[0, 0, :].reshape(RBS, 1)
    gid = lax.broadcasted_iota(jnp.int32, (RBS, BG), 1)
    onehot = (nb == gid).astype(jnp.float32)
    claim = claim_ref[...]
    ce = lax.dot_general(onehot, claim, (((1,), (0,)), ((), ())),
                         preferred_element_type=jnp.float32, precision=HIGH)
    dot = jnp.sum(ce * x, axis=1)
    na = jnp.sqrt(jnp.sum(ce * ce, axis=1))
    nx = jnp.sqrt(jnp.sum(x * x, axis=1))
    rel = dot / jnp.maximum(na * nx, 1e-8)
    h0 = x * rel[:, None]
    h1 = lax.dot_general(h0, w1_ref[...], (((1,), (0,)), ((), ())),
                         preferred_element_type=jnp.float32, precision=HIGH)
    h_ref[...] = h1
    as_ref[0, 0, :] = jnp.sum(h1 * avs_ref[...], axis=1)
    ad_ref[0, 0, :] = jnp.sum(h1 * avd_ref[...], axis=1)


def _tc1(x_p, nb3, claim, W1, att_src, att_dst):
    return pl.pallas_call(
        _tc1_body,
        grid=(RB,),
        in_specs=[
            pl.BlockSpec((RBS, DIN), lambda i: (i, 0)),
            pl.BlockSpec((1, 1, RBS), lambda i: (i, 0, 0)),
            pl.BlockSpec((BG, DIN), lambda i: (0, 0)),
            pl.BlockSpec((DIN, DH), lambda i: (0, 0)),
            pl.BlockSpec((1, DH), lambda i: (0, 0)),
            pl.BlockSpec((1, DH), lambda i: (0, 0)),
        ],
        out_specs=[
            pl.BlockSpec((RBS, DH), lambda i: (i, 0)),
            pl.BlockSpec((1, 1, RBS), lambda i: (i, 0, 0)),
            pl.BlockSpec((1, 1, RBS), lambda i: (i, 0, 0)),
        ],
        out_shape=[
            jax.ShapeDtypeStruct((NP, DH), jnp.float32),
            jax.ShapeDtypeStruct((RB, 1, RBS), jnp.float32),
            jax.ShapeDtypeStruct((RB, 1, RBS), jnp.float32),
        ],
    )(x_p, nb3, claim, W1, att_src, att_dst)


def _sca_body(src_h, dst_h, as_h, ad_h, ee_h, dpart_h,
              asrc_v, adst_v, srcv, dstv, eev, zbuf, denom_sp):
    cid = lax.axis_index("c")
    sid = lax.axis_index("s")
    wid = sid * NC + cid
    pltpu.sync_copy(as_h, asrc_v)
    pltpu.sync_copy(ad_h, adst_v)
    pltpu.sync_copy(src_h.at[wid], srcv)
    pltpu.sync_copy(dst_h.at[wid], dstv)
    z = jnp.zeros((L,), jnp.float32)

    def zb(i, carry):
        zbuf[pl.ds(i * L, L)] = z
        return carry
    lax.fori_loop(0, STRIP // L, zb, 0)
    pltpu.sync_copy(zbuf, denom_sp.at[pl.ds(sid * STRIP, STRIP)])
    plsc.subcore_barrier()

    def body(c, carry):
        for j in range(CH // L):
            s16 = srcv[c, pl.ds(j * L, L)]
            d16 = dstv[c, pl.ds(j * L, L)]
            t = plsc.load_gather(asrc_v, [s16]) + plsc.load_gather(adst_v, [d16])
            e = jnp.maximum(t, 0.2 * t)
            eev[c, pl.ds(j * L, L)] = jnp.exp(e)
        pltpu.sync_copy(eev.at[c], denom_sp.at[dstv.at[c]], add=True)
        return carry
    lax.fori_loop(0, NCHUNK, body, 0)
    pltpu.sync_copy(eev, ee_h.at[wid])
    plsc.subcore_barrier()
    pltpu.sync_copy(denom_sp.at[pl.ds(sid * STRIP, STRIP)],
                    dpart_h.at[cid, pl.ds(sid * STRIP, STRIP)])


def _sca(src2, dst2, a_src, a_dst):
    mesh = plsc.VectorSubcoreMesh(core_axis_name="c", subcore_axis_name="s", num_cores=NC, num_subcores=NS)
    f = pl.kernel(
        _sca_body,
        out_type=[
            jax.ShapeDtypeStruct((NW, NCHUNK, CH), jnp.float32),
            jax.ShapeDtypeStruct((NC, NP), jnp.float32),
        ],
        mesh=mesh,
        compiler_params=pltpu.CompilerParams(needs_layout_passes=False),
        scratch_types=[
            pltpu.VMEM((NP,), jnp.float32),
            pltpu.VMEM((NP,), jnp.float32),
            pltpu.VMEM((NCHUNK, CH), jnp.int32),
            pltpu.VMEM((NCHUNK, CH), jnp.int32),
            pltpu.VMEM((NCHUNK, CH), jnp.float32),
            pltpu.VMEM((STRIP,), jnp.float32),
            pltpu.VMEM_SHARED((NP,), jnp.float32),
        ],
    )
    return f(src2, dst2, a_src, a_dst)


C2 = 128
NCH2 = ET // C2


def _scb_body(src_h, dst_h, ee_h, dp_h, h_h, part_h,
              inv_v, d1s, rowsA, rowsB, iA, iB, dA, dB, eA, eB,
              gsA, gsB, ssA, ssB, fsA, fsB, out_sp):
    cid = lax.axis_index("c")
    sid = lax.axis_index("s")
    wid = sid * NC + cid
    pltpu.sync_copy(dp_h.at[0], inv_v)

    def invt(t, carry):
        pltpu.sync_copy(dp_h.at[1].at[pl.ds(t * STRIP, STRIP)], d1s)

        def invb(k, c2):
            s = pl.ds(t * STRIP + k * L, L)
            inv_v[s] = 1.0 / (inv_v[s] + d1s[pl.ds(k * L, L)] + 1e-16)
            return c2
        lax.fori_loop(0, STRIP // L, invb, 0)
        return carry
    lax.fori_loop(0, NS, invt, 0)

    z = jnp.zeros((L,), jnp.float32)

    def zrow(i, carry):
        for j in range(DH // L):
            rowsA[i, pl.ds(j * L, L)] = z
        return carry
    lax.fori_loop(0, C2, zrow, 0)

    def zstrip(k, carry):
        pltpu.sync_copy(rowsA, out_sp.at[pl.ds(sid * STRIP + k * C2, C2)])
        return carry
    lax.fori_loop(0, STRIP // C2, zstrip, 0)
    plsc.subcore_barrier()

    def _sl(c):
        return pl.ds(pl.multiple_of(c * C2, C2), C2)

    def fetch(c, ibuf, dbuf, ebuf, fsem):
        pltpu.async_copy(src_h.at[wid].at[_sl(c)], ibuf, fsem)
        pltpu.async_copy(dst_h.at[wid].at[_sl(c)], dbuf, fsem)
        pltpu.async_copy(ee_h.at[wid].at[_sl(c)], ebuf, fsem)

    def fetch_wait(c, ibuf, dbuf, ebuf, fsem):
        pltpu.make_async_copy(src_h.at[wid].at[_sl(c)], ibuf, fsem).wait()
        pltpu.make_async_copy(dst_h.at[wid].at[_sl(c)], dbuf, fsem).wait()
        pltpu.make_async_copy(ee_h.at[wid].at[_sl(c)], ebuf, fsem).wait()

    def compute(dbuf, ebuf, rbuf):
        def scale_j(j, carry):
            d16 = dbuf[pl.ds(j * L, L)]
            a16 = ebuf[pl.ds(j * L, L)] * plsc.load_gather(inv_v, [d16])
            for k in range(L):
                s = a16[k]
                i = j * L + k
                for jj in range(DH // L):
                    sl = pl.ds(jj * L, L)
                    rbuf[i, sl] = rbuf[i, sl] * s
            return carry
        lax.fori_loop(0, C2 // L, scale_j, 0)

    fetch(0, iA, dA, eA, fsA)
    fetch_wait(0, iA, dA, eA, fsA)
    pltpu.async_copy(h_h.at[iA], rowsA, gsA)

    def pair(g, carry):
        cB = 2 * g + 1
        @pl.when(g > 0)
        def _wb():
            pltpu.make_async_copy(rowsB, out_sp.at[dB], ssB).wait()
        fetch(cB, iB, dB, eB, fsB)
        pltpu.make_async_copy(h_h.at[iA], rowsA, gsA).wait()
        compute(dA, eA, rowsA)
        pltpu.async_copy(rowsA, out_sp.at[dA], ssA, add=True)
        fetch_wait(cB, iB, dB, eB, fsB)
        pltpu.async_copy(h_h.at[iB], rowsB, gsB)

        @pl.when(g < NCH2 // 2 - 1)
        def _na():
            pltpu.make_async_copy(rowsA, out_sp.at[dA], ssA).wait()
            fetch(2 * g + 2, iA, dA, eA, fsA)
        pltpu.make_async_copy(h_h.at[iB], rowsB, gsB).wait()
        compute(dB, eB, rowsB)
        pltpu.async_copy(rowsB, out_sp.at[dB], ssB, add=True)

        @pl.when(g < NCH2 // 2 - 1)
        def _ga():
            fetch_wait(2 * g + 2, iA, dA, eA, fsA)
            pltpu.async_copy(h_h.at[iA], rowsA, gsA)
        return carry
    lax.fori_loop(0, NCH2 // 2, pair, 0)
    last = NCH2 - 1
    pltpu.make_async_copy(rowsA, out_sp.at[dA], ssA).wait()
    fetch(last, iA, dA, eA, fsA)
    fetch_wait(last, iA, dA, eA, fsA)
    pltpu.async_copy(h_h.at[iA], rowsA, gsA)
    pltpu.make_async_copy(h_h.at[iA], rowsA, gsA).wait()
    compute(dA, eA, rowsA)
    pltpu.async_copy(rowsA, out_sp.at[dA], ssA, add=True)
    pltpu.make_async_copy(rowsA, out_sp.at[dA], ssA).wait()
    pltpu.make_async_copy(rowsB, out_sp.at[dB], ssB).wait()
    plsc.subcore_barrier()
    pltpu.sync_copy(out_sp.at[pl.ds(sid * STRIP, STRIP)],
                    part_h.at[cid, pl.ds(sid * STRIP, STRIP)])


def _scb(src2, dst2, ee, dparts, h):
    mesh = plsc.VectorSubcoreMesh(core_axis_name="c", subcore_axis_name="s", num_cores=NC, num_subcores=NS)
    f = pl.kernel(
        _scb_body,
        out_type=jax.ShapeDtypeStruct((NC, NP, DH), jnp.float32),
        mesh=mesh,
        compiler_params=pltpu.CompilerParams(needs_layout_passes=False),
        scratch_types=[
            pltpu.VMEM((NP,), jnp.float32),
            pltpu.VMEM((STRIP,), jnp.float32),
            pltpu.VMEM((C2, DH), jnp.float32),
            pltpu.VMEM((C2, DH), jnp.float32),
            pltpu.VMEM((C2,), jnp.int32),
            pltpu.VMEM((C2,), jnp.int32),
            pltpu.VMEM((C2,), jnp.int32),
            pltpu.VMEM((C2,), jnp.int32),
            pltpu.VMEM((C2,), jnp.float32),
            pltpu.VMEM((C2,), jnp.float32),
            pltpu.SemaphoreType.DMA,
            pltpu.SemaphoreType.DMA,
            pltpu.SemaphoreType.DMA,
            pltpu.SemaphoreType.DMA,
            pltpu.SemaphoreType.DMA,
            pltpu.SemaphoreType.DMA,
            pltpu.VMEM_SHARED((NP, DH), jnp.float32),
        ],
    )
    return f(src2, dst2, ee, dparts, h)


def _tc2a_body(p_ref, b_ref, o_ref, s_ref, q_ref):
    i = pl.program_id(0)
    o = p_ref[0] + p_ref[1] + b_ref[...]
    o_ref[...] = o
    rid = lax.broadcasted_iota(jnp.int32, (RBS, 1), 0) + i * RBS
    m = (rid < N).astype(jnp.float32)
    om = o * m
    s_ref[0, :, :] = jnp.sum(om, axis=0, keepdims=True)
    q_ref[0, :, :] = jnp.sum(om * o, axis=0, keepdims=True)


def _tc2a(parts, b1):
    return pl.pallas_call(
        _tc2a_body,
        grid=(RB,),
        in_specs=[
            pl.BlockSpec((2, RBS, DH), lambda i: (0, i, 0)),
            pl.BlockSpec((1, DH), lambda i: (0, 0)),
        ],
        out_specs=[
            pl.BlockSpec((RBS, DH), lambda i: (i, 0)),
            pl.BlockSpec((1, 1, DH), lambda i: (i, 0, 0)),
            pl.BlockSpec((1, 1, DH), lambda i: (i, 0, 0)),
        ],
        out_shape=[
            jax.ShapeDtypeStruct((NP, DH), jnp.float32),
            jax.ShapeDtypeStruct((RB, 1, DH), jnp.float32),
            jax.ShapeDtypeStruct((RB, 1, DH), jnp.float32),
        ],
    )(parts, b1)


def _tc2b_body(o_ref, mu_ref, is_ref, g_ref, be_ref, w2_ref, avs_ref, avd_ref,
               h_ref, as_ref, ad_ref):
    o = o_ref[...]
    y = (o - mu_ref[...]) * is_ref[...] * g_ref[...] + be_ref[...]
    y = jnp.maximum(y, 0.0)
    h2 = lax.dot_general(y, w2_ref[...], (((1,), (0,)), ((), ())),
                         preferred_element_type=jnp.float32, precision=HIGH)
    h_ref[...] = h2
    as_ref[0, 0, :] = jnp.sum(h2 * avs_ref[...], axis=1)
    ad_ref[0, 0, :] = jnp.sum(h2 * avd_ref[...], axis=1)


def _tc2b(o1, mu, istd, gamma, beta, W2, att_src, att_dst):
    return pl.pallas_call(
        _tc2b_body,
        grid=(RB,),
        in_specs=[
            pl.BlockSpec((RBS, DH), lambda i: (i, 0)),
            pl.BlockSpec((1, DH), lambda i: (0, 0)),
            pl.BlockSpec((1, DH), lambda i: (0, 0)),
            pl.BlockSpec((1, DH), lambda i: (0, 0)),
            pl.BlockSpec((1, DH), lambda i: (0, 0)),
            pl.BlockSpec((DH, DH), lambda i: (0, 0)),
            pl.BlockSpec((1, DH), lambda i: (0, 0)),
            pl.BlockSpec((1, DH), lambda i: (0, 0)),
        ],
        out_specs=[
            pl.BlockSpec((RBS, DH), lambda i: (i, 0)),
            pl.BlockSpec((1, 1, RBS), lambda i: (i, 0, 0)),
            pl.BlockSpec((1, 1, RBS), lambda i: (i, 0, 0)),
        ],
        out_shape=[
            jax.ShapeDtypeStruct((NP, DH), jnp.float32),
            jax.ShapeDtypeStruct((RB, 1, RBS), jnp.float32),
            jax.ShapeDtypeStruct((RB, 1, RBS), jnp.float32),
        ],
    )(o1, mu, istd, gamma, beta, W2, att_src, att_dst)


def _tc3_body(p_ref, b_ref, nb_ref, claim_ref, wc1_ref, wc2_ref, bc_ref,
              sum_ref, cnt_ref, out_ref):
    i = pl.program_id(0)
    o = p_ref[0] + p_ref[1] + b_ref[...]
    h = jnp.maximum(o, 0.0)
    nb = nb_ref[0, 0, :].reshape(RBS, 1)
    gid = lax.broadcasted_iota(jnp.int32, (RBS, BG), 1)
    onehot = (nb == gid).astype(jnp.float32)

    @pl.when(i == 0)
    def _init():
        sum_ref[...] = jnp.zeros_like(sum_ref)
        cnt_ref[...] = jnp.zeros_like(cnt_ref)

    sum_ref[...] += lax.dot_general(onehot, h, (((0,), (0,)), ((), ())),
                                    preferred_element_type=jnp.float32,
                                    precision=HIGH)
    ones = jnp.ones((RBS, DH), jnp.float32)
    cnt_ref[...] += lax.dot_general(onehot, ones, (((0,), (0,)), ((), ())),
                                    preferred_element_type=jnp.float32,
                                    precision=HIGH)

    @pl.when(i == RB - 1)
    def _final():
        pooled = sum_ref[...] / jnp.maximum(cnt_ref[...], 1.0)
        r = lax.dot_general(pooled, wc1_ref[...], (((1,), (0,)), ((), ())),
                            preferred_element_type=jnp.float32, precision=HIGH)
        r += lax.dot_general(claim_ref[...], wc2_ref[...],
                             (((1,), (0,)), ((), ())),
                             preferred_element_type=jnp.float32, precision=HIGH)
        out_ref[...] = r + bc_ref[...]


def _tc3(parts, b2, nb3, claim, Wc1, Wc2, bc):
    return pl.pallas_call(
        _tc3_body,
        grid=(RB,),
        in_specs=[
            pl.BlockSpec((2, RBS, DH), lambda i: (0, i, 0)),
            pl.BlockSpec((1, DH), lambda i: (0, 0)),
            pl.BlockSpec((1, 1, RBS), lambda i: (i, 0, 0)),
            pl.BlockSpec((BG, DIN), lambda i: (0, 0)),
            pl.BlockSpec((DH, 1), lambda i: (0, 0)),
            pl.BlockSpec((DIN, 1), lambda i: (0, 0)),
            pl.BlockSpec((1, 1), lambda i: (0, 0)),
        ],
        out_specs=[
            pl.BlockSpec((BG, DH), lambda i: (0, 0)),
            pl.BlockSpec((BG, DH), lambda i: (0, 0)),
            pl.BlockSpec((BG, 1), lambda i: (0, 0)),
        ],
        out_shape=[
            jax.ShapeDtypeStruct((BG, DH), jnp.float32),
            jax.ShapeDtypeStruct((BG, DH), jnp.float32),
            jax.ShapeDtypeStruct((BG, 1), jnp.float32),
        ],
    )(parts, b2, nb3, claim, Wc1, Wc2, bc)


def kernel(claim_embeddings, x, edge_index, node_batch,
           W1, att_src1, att_dst1, b1,
           W2, att_src2, att_dst2, b2,
           bn_gamma, bn_beta, Wc, bc):
    loop = jnp.arange(N, dtype=jnp.int32)
    padi = jnp.full((EP - E,), N, jnp.int32)
    src = jnp.concatenate([edge_index[0], loop, padi]).reshape(NW, NCHUNK, CH)
    dst = jnp.concatenate([edge_index[1], loop, padi]).reshape(NW, NCHUNK, CH)
    x_p = jnp.pad(x, ((0, NP - N), (0, 0)))
    nb3 = jnp.pad(node_batch.astype(jnp.int32), (0, NP - N),
                  constant_values=BG).reshape(RB, 1, RBS)
    avs1 = att_src1.reshape(1, DH)
    avd1 = att_dst1.reshape(1, DH)
    avs2 = att_src2.reshape(1, DH)
    avd2 = att_dst2.reshape(1, DH)

    srcf = src.reshape(NW, ET)
    dstf = dst.reshape(NW, ET)

    h1, a_s, a_d = _tc1(x_p, nb3, claim_embeddings, W1, avs1, avd1)
    ee1, dp1 = _sca(src, dst, a_s.reshape(NP), a_d.reshape(NP))
    parts1 = _scb(srcf, dstf, ee1.reshape(NW, ET), dp1, h1)

    o1, ps, pq = _tc2a(parts1, b1.reshape(1, DH))
    s = jnp.sum(ps.reshape(RB, DH), axis=0)
    q = jnp.sum(pq.reshape(RB, DH), axis=0)
    mu = s / N
    var = q / N - mu * mu
    istd = 1.0 / jnp.sqrt(var + 1e-5)
    h2, a_s2, a_d2 = _tc2b(o1, mu.reshape(1, DH), istd.reshape(1, DH),
                           bn_gamma.reshape(1, DH), bn_beta.reshape(1, DH),
                           W2, avs2, avd2)

    ee2, dp2 = _sca(src, dst, a_s2.reshape(NP), a_d2.reshape(NP))
    parts2 = _scb(srcf, dstf, ee2.reshape(NW, ET), dp2, h2)

    _, _, out = _tc3(parts2, b2.reshape(1, DH), nb3, claim_embeddings,
                     Wc[:DH], Wc[DH:], bc.reshape(1, 1))
    return out.reshape(BG)


# speedup vs baseline: 29.0736x; 1.1373x over previous
import functools
import jax
import jax.numpy as jnp
from jax import lax
from jax.experimental import pallas as pl
from jax.experimental.pallas import tpu as pltpu
from jax.experimental.pallas import tpu_sc as plsc

N = 10000
NP = 10240
BG = 32
DIN = 768
DH = 128
E_RAW = 320000
E = E_RAW + N
NC, NS, L = 2, 16, 16
NW = NC * NS
CH = 128
NCHUNK = 81
ET = NCHUNK * CH
EP = NW * ET
STRIP = NP // NS
RB = 16
RBS = NP // RB
HIGH = lax.Precision.HIGHEST


def _tc1_body(x_ref, nb_ref, claim_ref, w1_ref, avs_ref, avd_ref,
              h_ref, as_ref, ad_ref):
    x = x_ref[...]
    nb = nb_ref[0, 0, :].reshape(RBS, 1)
    gid = lax.broadcasted_iota(jnp.int32, (RBS, BG), 1)
    onehot = (nb == gid).astype(jnp.float32)
    claim = claim_ref[...]
    ce = lax.dot_general(onehot, claim, (((1,), (0,)), ((), ())),
                         preferred_element_type=jnp.float32, precision=HIGH)
    dot = jnp.sum(ce * x, axis=1)
    na = jnp.sqrt(jnp.sum(ce * ce, axis=1))
    nx = jnp.sqrt(jnp.sum(x * x, axis=1))
    rel = dot / jnp.maximum(na * nx, 1e-8)
    h0 = x * rel[:, None]
    h1 = lax.dot_general(h0, w1_ref[...], (((1,), (0,)), ((), ())),
                         preferred_element_type=jnp.float32, precision=HIGH)
    h_ref[...] = h1
    as_ref[0, 0, :] = jnp.sum(h1 * avs_ref[...], axis=1)
    ad_ref[0, 0, :] = jnp.sum(h1 * avd_ref[...], axis=1)


def _tc1(x_p, nb3, claim, W1, att_src, att_dst):
    return pl.pallas_call(
        _tc1_body,
        grid=(RB,),
        in_specs=[
            pl.BlockSpec((RBS, DIN), lambda i: (i, 0)),
            pl.BlockSpec((1, 1, RBS), lambda i: (i, 0, 0)),
            pl.BlockSpec((BG, DIN), lambda i: (0, 0)),
            pl.BlockSpec((DIN, DH), lambda i: (0, 0)),
            pl.BlockSpec((1, DH), lambda i: (0, 0)),
            pl.BlockSpec((1, DH), lambda i: (0, 0)),
        ],
        out_specs=[
            pl.BlockSpec((RBS, DH), lambda i: (i, 0)),
            pl.BlockSpec((1, 1, RBS), lambda i: (i, 0, 0)),
            pl.BlockSpec((1, 1, RBS), lambda i: (i, 0, 0)),
        ],
        out_shape=[
            jax.ShapeDtypeStruct((NP, DH), jnp.float32),
            jax.ShapeDtypeStruct((RB, 1, RBS), jnp.float32),
            jax.ShapeDtypeStruct((RB, 1, RBS), jnp.float32),
        ],
    )(x_p, nb3, claim, W1, att_src, att_dst)


def _sca_body(src_h, dst_h, as_h, ad_h, ee_h, dpart_h,
              asrc_v, adst_v, srcv, dstv, eev, zbuf, denom_sp):
    cid = lax.axis_index("c")
    sid = lax.axis_index("s")
    wid = sid * NC + cid
    pltpu.sync_copy(as_h, asrc_v)
    pltpu.sync_copy(ad_h, adst_v)
    pltpu.sync_copy(src_h.at[wid], srcv)
    pltpu.sync_copy(dst_h.at[wid], dstv)
    z = jnp.zeros((L,), jnp.float32)

    def zb(i, carry):
        zbuf[pl.ds(i * L, L)] = z
        return carry
    lax.fori_loop(0, STRIP // L, zb, 0)
    pltpu.sync_copy(zbuf, denom_sp.at[pl.ds(sid * STRIP, STRIP)])
    plsc.subcore_barrier()

    def body(c, carry):
        for j in range(CH // L):
            s16 = srcv[c, pl.ds(j * L, L)]
            d16 = dstv[c, pl.ds(j * L, L)]
            t = plsc.load_gather(asrc_v, [s16]) + plsc.load_gather(adst_v, [d16])
            e = jnp.maximum(t, 0.2 * t)
            eev[c, pl.ds(j * L, L)] = jnp.exp(e)
        pltpu.sync_copy(eev.at[c], denom_sp.at[dstv.at[c]], add=True)
        return carry
    lax.fori_loop(0, NCHUNK, body, 0)
    pltpu.sync_copy(eev, ee_h.at[wid])
    plsc.subcore_barrier()
    pltpu.sync_copy(denom_sp.at[pl.ds(sid * STRIP, STRIP)],
                    dpart_h.at[cid, pl.ds(sid * STRIP, STRIP)])


def _sca(src2, dst2, a_src, a_dst):
    mesh = plsc.VectorSubcoreMesh(core_axis_name="c", subcore_axis_name="s", num_cores=NC, num_subcores=NS)
    f = pl.kernel(
        _sca_body,
        out_type=[
            jax.ShapeDtypeStruct((NW, NCHUNK, CH), jnp.float32),
            jax.ShapeDtypeStruct((NC, NP), jnp.float32),
        ],
        mesh=mesh,
        compiler_params=pltpu.CompilerParams(needs_layout_passes=False),
        scratch_types=[
            pltpu.VMEM((NP,), jnp.float32),
            pltpu.VMEM((NP,), jnp.float32),
            pltpu.VMEM((NCHUNK, CH), jnp.int32),
            pltpu.VMEM((NCHUNK, CH), jnp.int32),
            pltpu.VMEM((NCHUNK, CH), jnp.float32),
            pltpu.VMEM((STRIP,), jnp.float32),
            pltpu.VMEM_SHARED((NP,), jnp.float32),
        ],
    )
    return f(src2, dst2, a_src, a_dst)


C2 = 64
NCH2 = ET // C2
QS = NCH2 // 6


def _scb_body(src_h, dst_h, ee_h, dp_h, h_h, part_h, *refs):
    (inv_v, d1s) = refs[0:2]
    rows = refs[2:5]
    ib = [refs[5 + 2 * l: 7 + 2 * l] for l in range(3)]
    db = [refs[11 + 2 * l: 13 + 2 * l] for l in range(3)]
    eb = [refs[17 + 2 * l: 19 + 2 * l] for l in range(3)]
    gs = refs[23:26]
    ss = refs[26:29]
    fs = refs[29:32]
    out_sp = refs[32]
    cid = lax.axis_index("c")
    sid = lax.axis_index("s")
    wid = sid * NC + cid
    pltpu.sync_copy(dp_h.at[0], inv_v)

    def invt(t, carry):
        pltpu.sync_copy(dp_h.at[1].at[pl.ds(t * STRIP, STRIP)], d1s)

        def invb(k, c2):
            s = pl.ds(t * STRIP + k * L, L)
            inv_v[s] = 1.0 / (inv_v[s] + d1s[pl.ds(k * L, L)] + 1e-16)
            return c2
        lax.fori_loop(0, STRIP // L, invb, 0)
        return carry
    lax.fori_loop(0, NS, invt, 0)

    z = jnp.zeros((L,), jnp.float32)

    def zrow(i, carry):
        for j in range(DH // L):
            rows[0][i, pl.ds(j * L, L)] = z
        return carry
    lax.fori_loop(0, C2, zrow, 0)

    def zstrip(k, carry):
        pltpu.sync_copy(rows[0], out_sp.at[pl.ds(sid * STRIP + k * C2, C2)])
        return carry
    lax.fori_loop(0, STRIP // C2, zstrip, 0)
    plsc.subcore_barrier()

    def _sl(c):
        return pl.ds(pl.multiple_of(c * C2, C2), C2)

    def fetch(c, leg, p):
        pltpu.async_copy(src_h.at[wid].at[_sl(c)], ib[leg][p], fs[leg])
        pltpu.async_copy(dst_h.at[wid].at[_sl(c)], db[leg][p], fs[leg])
        pltpu.async_copy(ee_h.at[wid].at[_sl(c)], eb[leg][p], fs[leg])

    def fetch_wait(c, leg, p):
        pltpu.make_async_copy(src_h.at[wid].at[_sl(c)], ib[leg][p], fs[leg]).wait()
        pltpu.make_async_copy(dst_h.at[wid].at[_sl(c)], db[leg][p], fs[leg]).wait()
        pltpu.make_async_copy(ee_h.at[wid].at[_sl(c)], eb[leg][p], fs[leg]).wait()

    def compute(dbuf, ebuf, rbuf):
        def scale_j(j, carry):
            d16 = dbuf[pl.ds(j * L, L)]
            a16 = ebuf[pl.ds(j * L, L)] * plsc.load_gather(inv_v, [d16])
            for k in range(L):
                s = a16[k]
                i = j * L + k
                for jj in range(DH // L):
                    sl = pl.ds(jj * L, L)
                    rbuf[i, sl] = rbuf[i, sl] * s
            return carry
        lax.fori_loop(0, C2 // L, scale_j, 0)

    for c0 in range(3):
        fetch(c0, c0, 0)
    for c0 in range(2):
        fetch_wait(c0, c0, 0)
        pltpu.async_copy(h_h.at[ib[c0][0]], rows[c0], gs[c0])

    def macro(q, carry):
        c6 = q * 6
        for k in range(6):
            c = c6 + k
            a, pa = k % 3, k // 3
            b, pb = (k + 2) % 3, ((k + 2) // 3) % 2
            pf = ((k + 3) // 3) % 2
            if k == 0:
                @pl.when(q > 0)
                def _w0():
                    pltpu.make_async_copy(rows[b], out_sp.at[db[b][pb]], ss[b]).wait()
            else:
                pltpu.make_async_copy(rows[b], out_sp.at[db[b][pb]], ss[b]).wait()
            if k >= 4:
                @pl.when(q < QS - 1)
                def _g1():
                    fetch_wait(c + 2, b, pb)
                    pltpu.async_copy(h_h.at[ib[b][pb]], rows[b], gs[b])
            else:
                fetch_wait(c + 2, b, pb)
                pltpu.async_copy(h_h.at[ib[b][pb]], rows[b], gs[b])
            if k >= 3:
                @pl.when(q < QS - 1)
                def _f1():
                    fetch(c + 3, a, pf)
            else:
                fetch(c + 3, a, pf)
            pltpu.make_async_copy(h_h.at[ib[a][pa]], rows[a], gs[a]).wait()
            compute(db[a][pa], eb[a][pa], rows[a])
            pltpu.async_copy(rows[a], out_sp.at[db[a][pa]], ss[a], add=True)
        return carry
    lax.fori_loop(0, QS, macro, 0)
    pltpu.make_async_copy(rows[2], out_sp.at[db[2][1]], ss[2]).wait()
    plsc.subcore_barrier()
    pltpu.sync_copy(out_sp.at[pl.ds(sid * STRIP, STRIP)],
                    part_h.at[cid, pl.ds(sid * STRIP, STRIP)])


def _scb(src2, dst2, ee, dparts, h):
    mesh = plsc.VectorSubcoreMesh(core_axis_name="c", subcore_axis_name="s", num_cores=NC, num_subcores=NS)
    small_i = [pltpu.VMEM((C2,), jnp.int32) for _ in range(6)]
    small_d = [pltpu.VMEM((C2,), jnp.int32) for _ in range(6)]
    small_e = [pltpu.VMEM((C2,), jnp.float32) for _ in range(6)]
    f = pl.kernel(
        _scb_body,
        out_type=jax.ShapeDtypeStruct((NC, NP, DH), jnp.float32),
        mesh=mesh,
        compiler_params=pltpu.CompilerParams(needs_layout_passes=False),
        scratch_types=(
            [pltpu.VMEM((NP,), jnp.float32), pltpu.VMEM((STRIP,), jnp.float32)]
            + [pltpu.VMEM((C2, DH), jnp.float32) for _ in range(3)]
            + small_i + small_d + small_e
            + [pltpu.SemaphoreType.DMA for _ in range(9)]
            + [pltpu.VMEM_SHARED((NP, DH), jnp.float32)]
        ),
    )
    return f(src2, dst2, ee, dparts, h)


def _tc2a_body(p_ref, b_ref, o_ref, s_ref, q_ref):
    i = pl.program_id(0)
    o = p_ref[0] + p_ref[1] + b_ref[...]
    o_ref[...] = o
    rid = lax.broadcasted_iota(jnp.int32, (RBS, 1), 0) + i * RBS
    m = (rid < N).astype(jnp.float32)
    om = o * m
    s_ref[0, :, :] = jnp.sum(om, axis=0, keepdims=True)
    q_ref[0, :, :] = jnp.sum(om * o, axis=0, keepdims=True)


def _tc2a(parts, b1):
    return pl.pallas_call(
        _tc2a_body,
        grid=(RB,),
        in_specs=[
            pl.BlockSpec((2, RBS, DH), lambda i: (0, i, 0)),
            pl.BlockSpec((1, DH), lambda i: (0, 0)),
        ],
        out_specs=[
            pl.BlockSpec((RBS, DH), lambda i: (i, 0)),
            pl.BlockSpec((1, 1, DH), lambda i: (i, 0, 0)),
            pl.BlockSpec((1, 1, DH), lambda i: (i, 0, 0)),
        ],
        out_shape=[
            jax.ShapeDtypeStruct((NP, DH), jnp.float32),
            jax.ShapeDtypeStruct((RB, 1, DH), jnp.float32),
            jax.ShapeDtypeStruct((RB, 1, DH), jnp.float32),
        ],
    )(parts, b1)


def _tc2b_body(o_ref, mu_ref, is_ref, g_ref, be_ref, w2_ref, avs_ref, avd_ref,
               h_ref, as_ref, ad_ref):
    o = o_ref[...]
    y = (o - mu_ref[...]) * is_ref[...] * g_ref[...] + be_ref[...]
    y = jnp.maximum(y, 0.0)
    h2 = lax.dot_general(y, w2_ref[...], (((1,), (0,)), ((), ())),
                         preferred_element_type=jnp.float32, precision=HIGH)
    h_ref[...] = h2
    as_ref[0, 0, :] = jnp.sum(h2 * avs_ref[...], axis=1)
    ad_ref[0, 0, :] = jnp.sum(h2 * avd_ref[...], axis=1)


def _tc2b(o1, mu, istd, gamma, beta, W2, att_src, att_dst):
    return pl.pallas_call(
        _tc2b_body,
        grid=(RB,),
        in_specs=[
            pl.BlockSpec((RBS, DH), lambda i: (i, 0)),
            pl.BlockSpec((1, DH), lambda i: (0, 0)),
            pl.BlockSpec((1, DH), lambda i: (0, 0)),
            pl.BlockSpec((1, DH), lambda i: (0, 0)),
            pl.BlockSpec((1, DH), lambda i: (0, 0)),
            pl.BlockSpec((DH, DH), lambda i: (0, 0)),
            pl.BlockSpec((1, DH), lambda i: (0, 0)),
            pl.BlockSpec((1, DH), lambda i: (0, 0)),
        ],
        out_specs=[
            pl.BlockSpec((RBS, DH), lambda i: (i, 0)),
            pl.BlockSpec((1, 1, RBS), lambda i: (i, 0, 0)),
            pl.BlockSpec((1, 1, RBS), lambda i: (i, 0, 0)),
        ],
        out_shape=[
            jax.ShapeDtypeStruct((NP, DH), jnp.float32),
            jax.ShapeDtypeStruct((RB, 1, RBS), jnp.float32),
            jax.ShapeDtypeStruct((RB, 1, RBS), jnp.float32),
        ],
    )(o1, mu, istd, gamma, beta, W2, att_src, att_dst)


def _tc3_body(p_ref, b_ref, nb_ref, claim_ref, wc1_ref, wc2_ref, bc_ref,
              sum_ref, cnt_ref, out_ref):
    i = pl.program_id(0)
    o = p_ref[0] + p_ref[1] + b_ref[...]
    h = jnp.maximum(o, 0.0)
    nb = nb_ref[0, 0, :].reshape(RBS, 1)
    gid = lax.broadcasted_iota(jnp.int32, (RBS, BG), 1)
    onehot = (nb == gid).astype(jnp.float32)

    @pl.when(i == 0)
    def _init():
        sum_ref[...] = jnp.zeros_like(sum_ref)
        cnt_ref[...] = jnp.zeros_like(cnt_ref)

    sum_ref[...] += lax.dot_general(onehot, h, (((0,), (0,)), ((), ())),
                                    preferred_element_type=jnp.float32,
                                    precision=HIGH)
    ones = jnp.ones((RBS, DH), jnp.float32)
    cnt_ref[...] += lax.dot_general(onehot, ones, (((0,), (0,)), ((), ())),
                                    preferred_element_type=jnp.float32,
                                    precision=HIGH)

    @pl.when(i == RB - 1)
    def _final():
        pooled = sum_ref[...] / jnp.maximum(cnt_ref[...], 1.0)
        r = lax.dot_general(pooled, wc1_ref[...], (((1,), (0,)), ((), ())),
                            preferred_element_type=jnp.float32, precision=HIGH)
        r += lax.dot_general(claim_ref[...], wc2_ref[...],
                             (((1,), (0,)), ((), ())),
                             preferred_element_type=jnp.float32, precision=HIGH)
        out_ref[...] = r + bc_ref[...]


def _tc3(parts, b2, nb3, claim, Wc1, Wc2, bc):
    return pl.pallas_call(
        _tc3_body,
        grid=(RB,),
        in_specs=[
            pl.BlockSpec((2, RBS, DH), lambda i: (0, i, 0)),
            pl.BlockSpec((1, DH), lambda i: (0, 0)),
            pl.BlockSpec((1, 1, RBS), lambda i: (i, 0, 0)),
            pl.BlockSpec((BG, DIN), lambda i: (0, 0)),
            pl.BlockSpec((DH, 1), lambda i: (0, 0)),
            pl.BlockSpec((DIN, 1), lambda i: (0, 0)),
            pl.BlockSpec((1, 1), lambda i: (0, 0)),
        ],
        out_specs=[
            pl.BlockSpec((BG, DH), lambda i: (0, 0)),
            pl.BlockSpec((BG, DH), lambda i: (0, 0)),
            pl.BlockSpec((BG, 1), lambda i: (0, 0)),
        ],
        out_shape=[
            jax.ShapeDtypeStruct((BG, DH), jnp.float32),
            jax.ShapeDtypeStruct((BG, DH), jnp.float32),
            jax.ShapeDtypeStruct((BG, 1), jnp.float32),
        ],
    )(parts, b2, nb3, claim, Wc1, Wc2, bc)


def kernel(claim_embeddings, x, edge_index, node_batch,
           W1, att_src1, att_dst1, b1,
           W2, att_src2, att_dst2, b2,
           bn_gamma, bn_beta, Wc, bc):
    loop = jnp.arange(N, dtype=jnp.int32)
    padi = jnp.full((EP - E,), N, jnp.int32)
    src = jnp.concatenate([edge_index[0], loop, padi]).reshape(NW, NCHUNK, CH)
    dst = jnp.concatenate([edge_index[1], loop, padi]).reshape(NW, NCHUNK, CH)
    x_p = jnp.pad(x, ((0, NP - N), (0, 0)))
    nb3 = jnp.pad(node_batch.astype(jnp.int32), (0, NP - N),
                  constant_values=BG).reshape(RB, 1, RBS)
    avs1 = att_src1.reshape(1, DH)
    avd1 = att_dst1.reshape(1, DH)
    avs2 = att_src2.reshape(1, DH)
    avd2 = att_dst2.reshape(1, DH)

    srcf = src.reshape(NW, ET)
    dstf = dst.reshape(NW, ET)

    h1, a_s, a_d = _tc1(x_p, nb3, claim_embeddings, W1, avs1, avd1)
    ee1, dp1 = _sca(src, dst, a_s.reshape(NP), a_d.reshape(NP))
    parts1 = _scb(srcf, dstf, ee1.reshape(NW, ET), dp1, h1)

    o1, ps, pq = _tc2a(parts1, b1.reshape(1, DH))
    s = jnp.sum(ps.reshape(RB, DH), axis=0)
    q = jnp.sum(pq.reshape(RB, DH), axis=0)
    mu = s / N
    var = q / N - mu * mu
    istd = 1.0 / jnp.sqrt(var + 1e-5)
    h2, a_s2, a_d2 = _tc2b(o1, mu.reshape(1, DH), istd.reshape(1, DH),
                           bn_gamma.reshape(1, DH), bn_beta.reshape(1, DH),
                           W2, avs2, avd2)

    ee2, dp2 = _sca(src, dst, a_s2.reshape(NP), a_d2.reshape(NP))
    parts2 = _scb(srcf, dstf, ee2.reshape(NW, ET), dp2, h2)

    _, _, out = _tc3(parts2, b2.reshape(1, DH), nb3, claim_embeddings,
                     Wc[:DH], Wc[DH:], bc.reshape(1, 1))
    return out.reshape(BG)
